# baseline (device time: 61682 ns/iter reference)
import os

import jax
import jax.numpy as jnp
from jax import lax
from jax.experimental import pallas as pl
from jax.experimental.pallas import tpu as pltpu

N_DEV = 32
M = 1024
N = 1024
CHUNK = M // N_DEV
SPLITS = int(os.environ.get("KERNEL_SPLITS", "2"))
HN = N // SPLITS

_VARIANT = os.environ.get("KERNEL_VARIANT", "full")


def _gelu(y):
    c = 0.7978845608028654
    return 0.5 * y * (1.0 + jnp.tanh(c * (y + 0.044715 * y * y * y)))


def kernel(x, w_mat):
    def body(x_ref, w_ref, out_ref, acc16_ref, out16_ref, rs_buf,
             rs_send_sems, rs_recv_sems, ag_send_sems, ag_recv_sems):
        me = lax.axis_index("i")

        barrier_sem = pltpu.get_barrier_semaphore()
        for k in range(1, N_DEV):
            pl.semaphore_signal(
                barrier_sem, inc=1,
                device_id=(lax.rem(me + k, N_DEV),),
                device_id_type=pl.DeviceIdType.MESH,
            )

        xb = x_ref[...].astype(jnp.bfloat16)
        wb = w_ref[...].astype(jnp.bfloat16)
        rs_rdmas = []
        for h in range(SPLITS):
            acc16_ref[:, pl.ds(h * HN, HN)] = jnp.dot(
                xb, wb[:, h * HN:(h + 1) * HN],
                preferred_element_type=jnp.float32,
            ).astype(jnp.bfloat16)
            rs_buf[h, 0] = acc16_ref[
                pl.ds(me * CHUNK, CHUNK), pl.ds(h * HN, HN)
            ]
            if h == 0:
                pl.semaphore_wait(barrier_sem, N_DEV - 1)
            if _VARIANT == "local_only":
                continue
            for k in range(1, N_DEV):
                peer = lax.rem(me + k, N_DEV)
                rdma = pltpu.make_async_remote_copy(
                    src_ref=acc16_ref.at[
                        pl.ds(peer * CHUNK, CHUNK), pl.ds(h * HN, HN)
                    ],
                    dst_ref=rs_buf.at[h, k],
                    send_sem=rs_send_sems.at[h, k],
                    recv_sem=rs_recv_sems.at[h, k],
                    device_id=(peer,),
                    device_id_type=pl.DeviceIdType.MESH,
                )
                rdma.start()
                rs_rdmas.append(rdma)

        ag_rdmas = []

        def rs_wait_accumulate_broadcast(h):
            def wait_k(k):
                src = lax.rem(me + N_DEV - k, N_DEV)
                rdma = pltpu.make_async_remote_copy(
                    src_ref=rs_buf.at[h, 0],
                    dst_ref=rs_buf.at[h, k],
                    send_sem=rs_send_sems.at[h, k],
                    recv_sem=rs_recv_sems.at[h, k],
                    device_id=(src,),
                    device_id_type=pl.DeviceIdType.MESH,
                )
                rdma.wait_recv()

            if _VARIANT != "local_only":
                for k in range(1, N_DEV // 2):
                    wait_k(k)
                own = jnp.sum(
                    rs_buf[h, : N_DEV // 2].astype(jnp.float32), axis=0
                )
                for k in range(N_DEV // 2, N_DEV):
                    wait_k(k)
                own = own + jnp.sum(
                    rs_buf[h, N_DEV // 2 :].astype(jnp.float32), axis=0
                )
            else:
                own = jnp.sum(rs_buf[h].astype(jnp.float32), axis=0)
            out16_ref[pl.ds(me * CHUNK, CHUNK), pl.ds(h * HN, HN)] = (
                _gelu(own).astype(jnp.bfloat16)
            )

            for k in range(1, N_DEV) if _VARIANT == "full" else []:
                peer = lax.rem(me + k, N_DEV)
                rdma = pltpu.make_async_remote_copy(
                    src_ref=out16_ref.at[
                        pl.ds(me * CHUNK, CHUNK), pl.ds(h * HN, HN)
                    ],
                    dst_ref=out16_ref.at[
                        pl.ds(me * CHUNK, CHUNK), pl.ds(h * HN, HN)
                    ],
                    send_sem=ag_send_sems.at[h, k],
                    recv_sem=ag_recv_sems.at[h, k],
                    device_id=(peer,),
                    device_id_type=pl.DeviceIdType.MESH,
                )
                rdma.start()
                ag_rdmas.append(rdma)

        def ag_wait_widen(h):
            for k in range(1, N_DEV) if _VARIANT == "full" else []:
                src = lax.rem(me + N_DEV - k, N_DEV)
                rdma = pltpu.make_async_remote_copy(
                    src_ref=out16_ref.at[
                        pl.ds(me * CHUNK, CHUNK), pl.ds(h * HN, HN)
                    ],
                    dst_ref=out16_ref.at[
                        pl.ds(src * CHUNK, CHUNK), pl.ds(h * HN, HN)
                    ],
                    send_sem=ag_send_sems.at[h, k],
                    recv_sem=ag_recv_sems.at[h, k],
                    device_id=(src,),
                    device_id_type=pl.DeviceIdType.MESH,
                )
                rdma.wait_recv()
            out_ref[:, pl.ds(h * HN, HN)] = (
                out16_ref[:, pl.ds(h * HN, HN)].astype(jnp.float32)
            )

        for h in range(SPLITS):
            rs_wait_accumulate_broadcast(h)
        for rdma in rs_rdmas:
            rdma.wait_send()
        for h in range(SPLITS):
            ag_wait_widen(h)
        for rdma in ag_rdmas:
            rdma.wait_send()

    return pl.pallas_call(
        body,
        out_shape=jax.ShapeDtypeStruct((M, N), jnp.float32),
        in_specs=[
            pl.BlockSpec(memory_space=pltpu.VMEM),
            pl.BlockSpec(memory_space=pltpu.VMEM),
        ],
        out_specs=pl.BlockSpec(memory_space=pltpu.VMEM),
        scratch_shapes=[
            pltpu.VMEM((M, N), jnp.bfloat16),
            pltpu.VMEM((M, N), jnp.bfloat16),
            pltpu.VMEM((SPLITS, N_DEV, CHUNK, HN), jnp.bfloat16),
            pltpu.SemaphoreType.DMA((SPLITS, N_DEV)),
            pltpu.SemaphoreType.DMA((SPLITS, N_DEV)),
            pltpu.SemaphoreType.DMA((SPLITS, N_DEV)),
            pltpu.SemaphoreType.DMA((SPLITS, N_DEV)),
        ],
        compiler_params=pltpu.CompilerParams(collective_id=0),
    )(x, w_mat)


# device time: 57413 ns/iter; 1.0744x vs baseline; 1.0744x over previous
import os

import jax
import jax.numpy as jnp
from jax import lax
from jax.experimental import pallas as pl
from jax.experimental.pallas import tpu as pltpu

N_DEV = 32
M = 1024
N = 1024
CHUNK = M // N_DEV
SPLITS = int(os.environ.get("KERNEL_SPLITS", "2"))
HN = N // SPLITS

_VARIANT = os.environ.get("KERNEL_VARIANT", "full")


def _gelu(y):
    c = 0.7978845608028654
    return 0.5 * y * (1.0 + jnp.tanh(c * (y + 0.044715 * y * y * y)))


def kernel(x, w_mat):
    def body(x_ref, w_ref, out_ref, acc16_ref, out16_ref, rs_buf,
             rs_send_sems, rs_recv_sems, ag_send_sems, ag_recv_sems):
        me = lax.axis_index("i")

        barrier_sem = pltpu.get_barrier_semaphore()
        for k in range(1, N_DEV):
            pl.semaphore_signal(
                barrier_sem, inc=1,
                device_id=(lax.rem(me + k, N_DEV),),
                device_id_type=pl.DeviceIdType.MESH,
            )

        xb = x_ref[...].astype(jnp.bfloat16)
        wb = w_ref[...].astype(jnp.bfloat16)
        rs_rdmas = []
        for h in range(SPLITS):
            acc16_ref[:, pl.ds(h * HN, HN)] = jnp.dot(
                xb, wb[:, h * HN:(h + 1) * HN],
                preferred_element_type=jnp.float32,
            ).astype(jnp.bfloat16)
            rs_buf[h, 0] = acc16_ref[
                pl.ds(me * CHUNK, CHUNK), pl.ds(h * HN, HN)
            ]
            if h == 0:
                pl.semaphore_wait(barrier_sem, N_DEV - 1)
            if _VARIANT == "local_only":
                continue
            for k in range(1, N_DEV):
                peer = lax.rem(me + k, N_DEV)
                rdma = pltpu.make_async_remote_copy(
                    src_ref=acc16_ref.at[
                        pl.ds(peer * CHUNK, CHUNK), pl.ds(h * HN, HN)
                    ],
                    dst_ref=rs_buf.at[h, k],
                    send_sem=rs_send_sems.at[h, k],
                    recv_sem=rs_recv_sems.at[h, k],
                    device_id=(peer,),
                    device_id_type=pl.DeviceIdType.MESH,
                )
                rdma.start()
                rs_rdmas.append(rdma)

        ag_rdmas = []

        def rs_wait_accumulate_broadcast(h):
            def wait_k(k):
                src = lax.rem(me + N_DEV - k, N_DEV)
                rdma = pltpu.make_async_remote_copy(
                    src_ref=rs_buf.at[h, 0],
                    dst_ref=rs_buf.at[h, k],
                    send_sem=rs_send_sems.at[h, k],
                    recv_sem=rs_recv_sems.at[h, k],
                    device_id=(src,),
                    device_id_type=pl.DeviceIdType.MESH,
                )
                rdma.wait_recv()

            if _VARIANT != "local_only":
                for k in range(1, N_DEV):
                    wait_k(k)
            own = jnp.sum(rs_buf[h].astype(jnp.float32), axis=0)
            out16_ref[pl.ds(me * CHUNK, CHUNK), pl.ds(h * HN, HN)] = (
                _gelu(own).astype(jnp.bfloat16)
            )

            for k in range(1, N_DEV) if _VARIANT == "full" else []:
                peer = lax.rem(me + k, N_DEV)
                rdma = pltpu.make_async_remote_copy(
                    src_ref=out16_ref.at[
                        pl.ds(me * CHUNK, CHUNK), pl.ds(h * HN, HN)
                    ],
                    dst_ref=out16_ref.at[
                        pl.ds(me * CHUNK, CHUNK), pl.ds(h * HN, HN)
                    ],
                    send_sem=ag_send_sems.at[h, k],
                    recv_sem=ag_recv_sems.at[h, k],
                    device_id=(peer,),
                    device_id_type=pl.DeviceIdType.MESH,
                )
                rdma.start()
                ag_rdmas.append(rdma)

        def ag_wait_widen(h):
            for k in range(1, N_DEV) if _VARIANT == "full" else []:
                src = lax.rem(me + N_DEV - k, N_DEV)
                rdma = pltpu.make_async_remote_copy(
                    src_ref=out16_ref.at[
                        pl.ds(me * CHUNK, CHUNK), pl.ds(h * HN, HN)
                    ],
                    dst_ref=out16_ref.at[
                        pl.ds(src * CHUNK, CHUNK), pl.ds(h * HN, HN)
                    ],
                    send_sem=ag_send_sems.at[h, k],
                    recv_sem=ag_recv_sems.at[h, k],
                    device_id=(src,),
                    device_id_type=pl.DeviceIdType.MESH,
                )
                rdma.wait_recv()
            out_ref[:, pl.ds(h * HN, HN)] = (
                out16_ref[:, pl.ds(h * HN, HN)].astype(jnp.float32)
            )

        for h in range(SPLITS):
            rs_wait_accumulate_broadcast(h)
        for rdma in rs_rdmas:
            rdma.wait_send()
        for h in range(SPLITS):
            ag_wait_widen(h)
        for rdma in ag_rdmas:
            rdma.wait_send()

    return pl.pallas_call(
        body,
        out_shape=jax.ShapeDtypeStruct((M, N), jnp.float32),
        in_specs=[
            pl.BlockSpec(memory_space=pltpu.VMEM),
            pl.BlockSpec(memory_space=pltpu.VMEM),
        ],
        out_specs=pl.BlockSpec(memory_space=pltpu.VMEM),
        scratch_shapes=[
            pltpu.VMEM((M, N), jnp.bfloat16),
            pltpu.VMEM((M, N), jnp.bfloat16),
            pltpu.VMEM((SPLITS, N_DEV, CHUNK, HN), jnp.bfloat16),
            pltpu.SemaphoreType.DMA((SPLITS, N_DEV)),
            pltpu.SemaphoreType.DMA((SPLITS, N_DEV)),
            pltpu.SemaphoreType.DMA((SPLITS, N_DEV)),
            pltpu.SemaphoreType.DMA((SPLITS, N_DEV)),
        ],
        compiler_params=pltpu.CompilerParams(collective_id=0),
    )(x, w_mat)
